# baseline (speedup 1.0000x reference)
import jax
import jax.numpy as jnp
from jax import lax
from jax.experimental import pallas as pl
from jax.experimental.pallas import tpu as pltpu
from jax.experimental.pallas import tpu_sc as plsc

EMBED = 8
LANES = 16
TILE = 128
NUM_CORES = 2
NUM_SUBCORES = 16
NUM_WORKERS = NUM_CORES * NUM_SUBCORES


def _copy_body(ut_hbm, it_hbm, out_u, out_i, sem_u, sem_i):
    wid = lax.axis_index("s") * NUM_CORES + lax.axis_index("c")
    ntiles = out_u.shape[0]
    per_w = -(-ntiles // NUM_WORKERS)
    last_w = (ntiles - 1) // per_w
    base = wid * per_w
    nmine = jnp.where(
        wid < last_w, per_w,
        jnp.where(wid == last_w, ntiles - last_w * per_w, 0))

    def fire(i, carry):
        k = base + i
        pltpu.async_copy(ut_hbm.at[:, pl.ds(k * TILE, TILE)], out_u.at[k],
                         sem_u)
        pltpu.async_copy(it_hbm.at[:, pl.ds(k * TILE, TILE)], out_i.at[k],
                         sem_i)
        return carry

    lax.fori_loop(0, nmine, fire, 0)

    def drain(i, carry):
        pltpu.make_async_copy(ut_hbm.at[:, pl.ds(0, TILE)], out_u.at[0],
                              sem_u).wait()
        pltpu.make_async_copy(it_hbm.at[:, pl.ds(0, TILE)], out_i.at[0],
                              sem_i).wait()
        return carry

    lax.fori_loop(0, nmine, drain, 0)


def _gather_body(chunk, max_row, users_hbm, items_hbm, tu_hbm, ti_hbm,
                 fu_hbm, fi_hbm, out_hbm, idx_u, idx_i, eidx_u, eidx_i,
                 gu, gi, out_v, tvm_u, tvm_i, sem_u, sem_i):
    wid = lax.axis_index("s") * NUM_CORES + lax.axis_index("c")
    base = wid * chunk
    pltpu.sync_copy(users_hbm.at[pl.ds(base, chunk)], idx_u)
    pltpu.sync_copy(items_hbm.at[pl.ds(base, chunk)], idx_i)
    pltpu.sync_copy(tu_hbm, tvm_u)
    pltpu.sync_copy(ti_hbm, tvm_i)

    groups = chunk // LANES

    def expand(g, carry):
        s = pl.ds(g * LANES, LANES)
        ru = jnp.minimum(idx_u[s], max_row)
        ri = jnp.minimum(idx_i[s], max_row)
        bu = ((ru >> 7) << 10) + (ru & 127)
        bi = ((ri >> 7) << 10) + (ri & 127)
        for d in range(EMBED):
            t = pl.ds(d * chunk + g * LANES, LANES)
            eidx_u[t] = bu + d * TILE
            eidx_i[t] = bi + d * TILE
        return carry

    lax.fori_loop(0, groups, expand, 0)

    cp_u = pltpu.async_copy(fu_hbm.at[eidx_u], gu, sem_u)
    cp_i = pltpu.async_copy(fi_hbm.at[eidx_i], gi, sem_i)
    cp_u.wait()
    cp_i.wait()

    def dot(g, carry):
        acc = None
        for d in range(EMBED):
            t = pl.ds(d * chunk + g * LANES, LANES)
            u = gu[t]
            v = gi[t]
            acc = u * v if acc is None else acc + u * v
        out_v[pl.ds(g * LANES, LANES)] = acc

        s = pl.ds(g * LANES, LANES)
        ru = idx_u[s]
        ri = idx_i[s]
        has_tail = jnp.max(jnp.maximum(ru, ri), axis=0) > max_row

        @pl.when(has_tail)
        def _():
            mu = ru > max_row
            mi = ri > max_row
            cu = jnp.where(mu, ru - (max_row + 1), 0)
            ci = jnp.where(mi, ri - (max_row + 1), 0)
            acc2 = None
            for d in range(EMBED):
                t = pl.ds(d * chunk + g * LANES, LANES)
                dsp = jnp.full((LANES,), d, jnp.int32)
                u = jnp.where(mu, plsc.load_gather(tvm_u, [dsp, cu]), gu[t])
                v = jnp.where(mi, plsc.load_gather(tvm_i, [dsp, ci]), gi[t])
                acc2 = u * v if acc2 is None else acc2 + u * v
            out_v[pl.ds(g * LANES, LANES)] = acc2

        return carry

    lax.fori_loop(0, groups, dot, 0)

    pltpu.sync_copy(out_v, out_hbm.at[pl.ds(base, chunk)])


def kernel(users, items, user_table, item_table):
    batch = users.shape[0]
    chunk = batch // NUM_WORKERS
    rows = user_table.shape[0]
    nfull = rows // TILE
    tail = rows - nfull * TILE
    ut = user_table.T
    it = item_table.T
    tail_u = ut[:, nfull * TILE:]
    tail_i = it[:, nfull * TILE:]
    mesh = plsc.VectorSubcoreMesh(core_axis_name="c", subcore_axis_name="s")
    params = pltpu.CompilerParams(needs_layout_passes=False)

    k1 = pl.kernel(
        _copy_body,
        mesh=mesh,
        compiler_params=params,
        out_type=(
            jax.ShapeDtypeStruct((nfull, EMBED, TILE), jnp.float32),
            jax.ShapeDtypeStruct((nfull, EMBED, TILE), jnp.float32),
        ),
        scratch_types=[
            pltpu.SemaphoreType.DMA,
            pltpu.SemaphoreType.DMA,
        ],
    )

    def gather_body(*refs):
        _gather_body(chunk, nfull * TILE - 1, *refs)

    k2 = pl.kernel(
        gather_body,
        mesh=mesh,
        compiler_params=params,
        out_type=jax.ShapeDtypeStruct((batch,), jnp.float32),
        scratch_types=[
            pltpu.VMEM((chunk,), jnp.int32),
            pltpu.VMEM((chunk,), jnp.int32),
            pltpu.VMEM((chunk * EMBED,), jnp.int32),
            pltpu.VMEM((chunk * EMBED,), jnp.int32),
            pltpu.VMEM((chunk * EMBED,), jnp.float32),
            pltpu.VMEM((chunk * EMBED,), jnp.float32),
            pltpu.VMEM((chunk,), jnp.float32),
            pltpu.VMEM((EMBED, tail), jnp.float32),
            pltpu.VMEM((EMBED, tail), jnp.float32),
            pltpu.SemaphoreType.DMA,
            pltpu.SemaphoreType.DMA,
        ],
    )

    users = users.astype(jnp.int32)
    items = items.astype(jnp.int32)
    img_u, img_i = k1(ut, it)
    return k2(users, items, tail_u, tail_i,
              img_u.reshape(-1), img_i.reshape(-1))


# speedup vs baseline: 22.7074x; 22.7074x over previous
import jax
import jax.numpy as jnp
from jax import lax
from jax.experimental import pallas as pl
from jax.experimental.pallas import tpu as pltpu
from jax.experimental.pallas import tpu_sc as plsc

EMBED = 8
LANES = 16
NUM_CORES = 2
NUM_SUBCORES = 16
NUM_WORKERS = NUM_CORES * NUM_SUBCORES


def _dot_body(chunk, users_hbm, items_hbm, ut_hbm, it_hbm, out_hbm,
              idx_u, idx_i, out_v, stg_u, stg_i, sem_u, sem_i):
    wid = lax.axis_index("s") * NUM_CORES + lax.axis_index("c")
    base = wid * chunk
    pltpu.sync_copy(users_hbm.at[pl.ds(base, chunk)], idx_u)
    pltpu.sync_copy(items_hbm.at[pl.ds(base, chunk)], idx_i)

    groups = chunk // LANES
    lane = lax.iota(jnp.int32, LANES)

    def dot(g, carry):
        s = pl.ds(g * LANES, LANES)
        iu = idx_u[s]
        ii = idx_i[s]
        cps = []
        for j in range(LANES):
            cu = pl.multiple_of((iu[j] >> 7) * 128, 128)
            ci = pl.multiple_of((ii[j] >> 7) * 128, 128)
            cps.append(pltpu.async_copy(ut_hbm.at[:, pl.ds(cu, 128)],
                                        stg_u.at[j], sem_u))
            cps.append(pltpu.async_copy(it_hbm.at[:, pl.ds(ci, 128)],
                                        stg_i.at[j], sem_i))
        for cp in cps:
            cp.wait()
        cu127 = iu & 127
        ci127 = ii & 127
        acc = None
        for d in range(EMBED):
            dsp = jnp.full((LANES,), d, jnp.int32)
            u = plsc.load_gather(stg_u, [lane, dsp, cu127])
            v = plsc.load_gather(stg_i, [lane, dsp, ci127])
            acc = u * v if acc is None else acc + u * v
        out_v[s] = acc
        return carry

    lax.fori_loop(0, groups, dot, 0)

    pltpu.sync_copy(out_v, out_hbm.at[pl.ds(base, chunk)])


def kernel(users, items, user_table, item_table):
    batch = users.shape[0]
    chunk = batch // NUM_WORKERS
    ut = user_table.T
    it = item_table.T
    mesh = plsc.VectorSubcoreMesh(core_axis_name="c", subcore_axis_name="s")

    def body(*refs):
        _dot_body(chunk, *refs)

    k = pl.kernel(
        body,
        mesh=mesh,
        compiler_params=pltpu.CompilerParams(needs_layout_passes=False),
        out_type=jax.ShapeDtypeStruct((batch,), jnp.float32),
        scratch_types=[
            pltpu.VMEM((chunk,), jnp.int32),
            pltpu.VMEM((chunk,), jnp.int32),
            pltpu.VMEM((chunk,), jnp.float32),
            pltpu.VMEM((LANES, EMBED, 128), jnp.float32),
            pltpu.VMEM((LANES, EMBED, 128), jnp.float32),
            pltpu.SemaphoreType.DMA,
            pltpu.SemaphoreType.DMA,
        ],
    )
    return k(users.astype(jnp.int32), items.astype(jnp.int32), ut, it)
